# Initial kernel scaffold; baseline (speedup 1.0000x reference)
import jax
import jax.numpy as jnp
from jax.experimental import pallas as pl


def kernel(token_ids, position_ids, token_table, pos_table):
    raise NotImplementedError("write your pallas kernel here")


# speedup vs baseline: 1.0474x; 1.0474x over previous
import functools

import jax
import jax.numpy as jnp
from jax import lax
from jax.experimental import pallas as pl
from jax.experimental.pallas import tpu as pltpu
from jax.experimental.pallas import tpu_sc as plsc

_LANES = 16
_NUM_WORKERS = 32
_CHUNK = 16


def _sc_embed_call(n_rows, hidden):
    per_w = n_rows // _NUM_WORKERS
    n_chunks = per_w // _CHUNK
    mesh = plsc.VectorSubcoreMesh(core_axis_name="c", subcore_axis_name="s")

    @functools.partial(
        pl.kernel,
        mesh=mesh,
        out_type=jax.ShapeDtypeStruct((n_rows, hidden), jnp.float32),
        scratch_types=[
            pltpu.VMEM((per_w,), jnp.int32),
            pltpu.VMEM((per_w,), jnp.int32),
            pltpu.VMEM((_CHUNK, hidden), jnp.float32),
            pltpu.VMEM((_CHUNK, hidden), jnp.float32),
            pltpu.SemaphoreType.DMA,
            pltpu.SemaphoreType.DMA,
        ],
    )
    def sc_embed(tok_hbm, pos_hbm, ttab_hbm, ptab_hbm, out_hbm,
                 tidx_v, pidx_v, trows, prows, sem_t, sem_p):
        wid = lax.axis_index("s") * 2 + lax.axis_index("c")
        base = wid * per_w
        pltpu.sync_copy(tok_hbm.at[pl.ds(base, per_w)], tidx_v)
        pltpu.sync_copy(pos_hbm.at[pl.ds(base, per_w)], pidx_v)

        n_vec = hidden // _LANES

        for ci in range(n_chunks):
            off = ci * _CHUNK
            cp_t = pltpu.async_copy(
                ttab_hbm.at[tidx_v.at[pl.ds(off, _CHUNK)]], trows, sem_t)
            cp_p = pltpu.async_copy(
                ptab_hbm.at[pidx_v.at[pl.ds(off, _CHUNK)]], prows, sem_p)
            cp_t.wait()
            cp_p.wait()

            def add_body(j, _, trows=trows, prows=prows):
                col = j * _LANES
                for r in range(_CHUNK):
                    sl = pl.ds(col, _LANES)
                    trows[r, sl] = trows[r, sl] + prows[r, sl]
                return 0

            lax.fori_loop(0, n_vec, add_body, 0)
            pltpu.sync_copy(trows, out_hbm.at[pl.ds(base + off, _CHUNK)])

    return sc_embed


def kernel(token_ids, position_ids, token_table, pos_table):
    b, s = token_ids.shape
    _, hidden = token_table.shape
    n_rows = b * s
    tids = token_ids.reshape(n_rows).astype(jnp.int32)
    pids = position_ids.reshape(n_rows).astype(jnp.int32)
    out = _sc_embed_call(n_rows, hidden)(tids, pids, token_table, pos_table)
    return out.reshape(b, s, hidden)


# speedup vs baseline: 1.2682x; 1.2107x over previous
import functools

import jax
import jax.numpy as jnp
from jax import lax
from jax.experimental import pallas as pl
from jax.experimental.pallas import tpu as pltpu
from jax.experimental.pallas import tpu_sc as plsc

_LANES = 16
_NUM_WORKERS = 32
_CHUNK = 16


def _sc_embed_call(n_rows, hidden):
    per_w = n_rows // _NUM_WORKERS
    n_chunks = per_w // _CHUNK
    mesh = plsc.VectorSubcoreMesh(core_axis_name="c", subcore_axis_name="s")

    @functools.partial(
        pl.kernel,
        mesh=mesh,
        out_type=jax.ShapeDtypeStruct((n_rows, hidden), jnp.float32),
        scratch_types=[
            pltpu.VMEM((per_w,), jnp.int32),
            pltpu.VMEM((per_w,), jnp.int32),
            pltpu.VMEM((_CHUNK, hidden), jnp.float32),
            pltpu.VMEM((_CHUNK, hidden), jnp.float32),
            pltpu.VMEM((_CHUNK, hidden), jnp.float32),
            pltpu.VMEM((_CHUNK, hidden), jnp.float32),
            pltpu.SemaphoreType.DMA,
            pltpu.SemaphoreType.DMA,
            pltpu.SemaphoreType.DMA,
            pltpu.SemaphoreType.DMA,
            pltpu.SemaphoreType.DMA,
            pltpu.SemaphoreType.DMA,
        ],
    )
    def sc_embed(tok_hbm, pos_hbm, ttab_hbm, ptab_hbm, out_hbm,
                 tidx_v, pidx_v, trows0, prows0, trows1, prows1,
                 s_tg0, s_pg0, s_st0, s_tg1, s_pg1, s_st1):
        wid = lax.axis_index("s") * 2 + lax.axis_index("c")
        base = wid * per_w
        pltpu.sync_copy(tok_hbm.at[pl.ds(base, per_w)], tidx_v)
        pltpu.sync_copy(pos_hbm.at[pl.ds(base, per_w)], pidx_v)

        n_vec = hidden // _LANES
        bufs = [(trows0, prows0, s_tg0, s_pg0, s_st0),
                (trows1, prows1, s_tg1, s_pg1, s_st1)]
        gathers = [None, None]
        stores = [None, None]

        def start_gather(ci, b):
            off = ci * _CHUNK
            tb, pb, s_tg, s_pg, _ = bufs[b]
            g_t = pltpu.async_copy(
                ttab_hbm.at[tidx_v.at[pl.ds(off, _CHUNK)]], tb, s_tg)
            g_p = pltpu.async_copy(
                ptab_hbm.at[pidx_v.at[pl.ds(off, _CHUNK)]], pb, s_pg)
            gathers[b] = (g_t, g_p)

        start_gather(0, 0)
        for ci in range(n_chunks):
            cur = ci % 2
            nxt = 1 - cur
            if ci + 1 < n_chunks:
                if stores[nxt] is not None:
                    stores[nxt].wait()
                    stores[nxt] = None
                start_gather(ci + 1, nxt)
            g_t, g_p = gathers[cur]
            g_t.wait()
            g_p.wait()
            tb, pb, _, _, s_st = bufs[cur]

            def add_body(j, _, tb=tb, pb=pb):
                sl = pl.ds(j * _LANES, _LANES)
                for r in range(_CHUNK):
                    plsc.addupdate(tb.at[r, sl], pb[r, sl])
                return 0

            lax.fori_loop(0, n_vec, add_body, 0)
            stores[cur] = pltpu.async_copy(
                tb, out_hbm.at[pl.ds(base + ci * _CHUNK, _CHUNK)], s_st)
        for b in range(2):
            if stores[b] is not None:
                stores[b].wait()

    return sc_embed


def kernel(token_ids, position_ids, token_table, pos_table):
    b, s = token_ids.shape
    _, hidden = token_table.shape
    n_rows = b * s
    tids = token_ids.reshape(n_rows).astype(jnp.int32)
    pids = position_ids.reshape(n_rows).astype(jnp.int32)
    out = _sc_embed_call(n_rows, hidden)(tids, pids, token_table, pos_table)
    return out.reshape(b, s, hidden)
